# baseline (speedup 1.0000x reference)
import functools

import jax
import jax.numpy as jnp
import numpy as np
from jax import lax
from jax.experimental import pallas as pl
from jax.experimental.pallas import tpu as pltpu
from jax.experimental.pallas import tpu_sc as plsc

DROP_RATE = 0.15
NUM_ROWS = 16384
ROW_DIM = 64
NUM_KEEP = max(1, int(NUM_ROWS * (1.0 - DROP_RATE)))

NUM_CORES = 2
NUM_SUBCORES = 16
NUM_WORKERS = NUM_CORES * NUM_SUBCORES
ROWS_PER_WORKER = ROW_DIM // NUM_WORKERS

LANES = 16
SUBLANES = 8
TILE_MINOR = 128
IN_TILES = NUM_ROWS // TILE_MINOR
OUT_TILES = -(-NUM_KEEP // TILE_MINOR)
KEEP_PAD = OUT_TILES * TILE_MINOR
NUM_STEPS = KEEP_PAD // LANES


@functools.lru_cache(maxsize=1)
def _gather_indices() -> np.ndarray:
    with jax.ensure_compile_time_eval():
        perm = jax.random.permutation(jax.random.key(42), NUM_ROWS)
    idx = np.asarray(perm)[:NUM_KEEP].astype(np.int32)
    return np.concatenate([idx, np.full(KEEP_PAD - NUM_KEEP, idx[-1], np.int32)])


@functools.lru_cache(maxsize=1)
def _build_gather():
    mesh = plsc.VectorSubcoreMesh(core_axis_name="c", subcore_axis_name="s")

    @functools.partial(
        pl.kernel,
        out_type=jax.ShapeDtypeStruct(
            (ROW_DIM // SUBLANES, OUT_TILES, SUBLANES, TILE_MINOR), jnp.float32
        ),
        mesh=mesh,
        compiler_params=pltpu.CompilerParams(
            use_tc_tiling_on_sc=False, needs_layout_passes=False
        ),
        scratch_types=[
            pltpu.VMEM((KEEP_PAD,), jnp.int32),
            pltpu.VMEM((IN_TILES, ROWS_PER_WORKER, TILE_MINOR), jnp.float32),
            pltpu.VMEM((OUT_TILES, ROWS_PER_WORKER, TILE_MINOR), jnp.float32),
            pltpu.SemaphoreType.DMA,
        ],
    )
    def _gather_cols(z_hbm, idx_hbm, y_hbm, idx_v, tbl_v, res_v, sem):
        wid = lax.axis_index("s") * NUM_CORES + lax.axis_index("c")
        row0 = wid * ROWS_PER_WORKER
        a = row0 // SUBLANES
        rr = row0 % SUBLANES
        cp_idx = pltpu.async_copy(idx_hbm, idx_v, sem)
        cp_tbl = pltpu.async_copy(
            z_hbm.at[a, :, pl.ds(rr, ROWS_PER_WORKER), :], tbl_v, sem
        )
        cp_idx.wait()
        cp_tbl.wait()

        steps_per_tile = TILE_MINOR // LANES

        def tile_step(q):
            base = q * TILE_MINOR
            for j in range(steps_per_tile):
                cols = idx_v[pl.ds(base + j * LANES, LANES)]
                b = cols >> 7
                c = cols & (TILE_MINOR - 1)
                for r in range(ROWS_PER_WORKER):
                    rows = jnp.full((LANES,), r, jnp.int32)
                    res_v[q, r, pl.ds(j * LANES, LANES)] = plsc.load_gather(
                        tbl_v, [b, rows, c]
                    )

        plsc.parallel_loop(0, OUT_TILES, 1, unroll=2, carry=None)(tile_step)

        pltpu.sync_copy(res_v, y_hbm.at[a, :, pl.ds(rr, ROWS_PER_WORKER), :])

    return _gather_cols


def kernel(instances):
    z = instances.T.reshape(
        ROW_DIM // SUBLANES, SUBLANES, IN_TILES, TILE_MINOR
    ).transpose(0, 2, 1, 3)
    y = _build_gather()(z, jnp.asarray(_gather_indices()))
    out_t = y.transpose(0, 2, 1, 3).reshape(ROW_DIM, KEEP_PAD)
    return out_t[:, :NUM_KEEP].T


# speedup vs baseline: 1.0032x; 1.0032x over previous
import functools

import jax
import jax.numpy as jnp
import numpy as np
from jax import lax
from jax.experimental import pallas as pl
from jax.experimental.pallas import tpu as pltpu
from jax.experimental.pallas import tpu_sc as plsc

DROP_RATE = 0.15
NUM_ROWS = 16384
ROW_DIM = 64
NUM_KEEP = max(1, int(NUM_ROWS * (1.0 - DROP_RATE)))

NUM_CORES = 2
NUM_SUBCORES = 16
NUM_WORKERS = NUM_CORES * NUM_SUBCORES
ROWS_PER_WORKER = ROW_DIM // NUM_WORKERS

LANES = 16
SUBLANES = 8
TILE_MINOR = 128
IN_TILES = NUM_ROWS // TILE_MINOR
OUT_TILES = -(-NUM_KEEP // TILE_MINOR)
KEEP_PAD = OUT_TILES * TILE_MINOR
NUM_STEPS = KEEP_PAD // LANES


@functools.lru_cache(maxsize=1)
def _gather_indices() -> np.ndarray:
    with jax.ensure_compile_time_eval():
        perm = jax.random.permutation(jax.random.key(42), NUM_ROWS)
    idx = np.asarray(perm)[:NUM_KEEP].astype(np.int32)
    return np.concatenate([idx, np.full(KEEP_PAD - NUM_KEEP, idx[-1], np.int32)])


@functools.lru_cache(maxsize=1)
def _build_gather():
    mesh = plsc.VectorSubcoreMesh(core_axis_name="c", subcore_axis_name="s")

    @functools.partial(
        pl.kernel,
        out_type=jax.ShapeDtypeStruct(
            (ROW_DIM // SUBLANES, OUT_TILES, SUBLANES, TILE_MINOR), jnp.float32
        ),
        mesh=mesh,
        compiler_params=pltpu.CompilerParams(
            use_tc_tiling_on_sc=False, needs_layout_passes=False
        ),
        scratch_types=[
            pltpu.VMEM((KEEP_PAD,), jnp.int32),
            pltpu.VMEM((IN_TILES, ROWS_PER_WORKER, TILE_MINOR), jnp.float32),
            pltpu.VMEM((OUT_TILES, ROWS_PER_WORKER, TILE_MINOR), jnp.float32),
            pltpu.SemaphoreType.DMA,
        ],
    )
    def _gather_cols(z_hbm, idx_hbm, y_hbm, idx_v, tbl_v, res_v, sem):
        wid = lax.axis_index("s") * NUM_CORES + lax.axis_index("c")
        row0 = wid * ROWS_PER_WORKER
        a = row0 // SUBLANES
        rr = row0 % SUBLANES
        cp_idx = pltpu.async_copy(idx_hbm, idx_v, sem)
        cp_tbl = pltpu.async_copy(
            z_hbm.at[a, :, pl.ds(rr, ROWS_PER_WORKER), :], tbl_v, sem
        )
        cp_idx.wait()
        cp_tbl.wait()

        steps_per_tile = TILE_MINOR // LANES

        def tile_step(q):
            base = q * TILE_MINOR
            for j in range(steps_per_tile):
                cols = idx_v[pl.ds(base + j * LANES, LANES)]
                b = cols >> 7
                c = cols & (TILE_MINOR - 1)
                for r in range(ROWS_PER_WORKER):
                    rows = jnp.full((LANES,), r, jnp.int32)
                    res_v[q, r, pl.ds(j * LANES, LANES)] = plsc.load_gather(
                        tbl_v, [b, rows, c]
                    )

        plsc.parallel_loop(0, OUT_TILES, 1, unroll=4, carry=None)(tile_step)

        pltpu.sync_copy(res_v, y_hbm.at[a, :, pl.ds(rr, ROWS_PER_WORKER), :])

    return _gather_cols


def kernel(instances):
    z = instances.T.reshape(
        ROW_DIM // SUBLANES, SUBLANES, IN_TILES, TILE_MINOR
    ).transpose(0, 2, 1, 3)
    y = _build_gather()(z, jnp.asarray(_gather_indices()))
    out_t = y.transpose(0, 2, 1, 3).reshape(ROW_DIM, KEEP_PAD)
    return out_t[:, :NUM_KEEP].T


# speedup vs baseline: 1.0289x; 1.0256x over previous
import functools

import jax
import jax.numpy as jnp
import numpy as np
from jax import lax
from jax.experimental import pallas as pl
from jax.experimental.pallas import tpu as pltpu
from jax.experimental.pallas import tpu_sc as plsc

DROP_RATE = 0.15
NUM_ROWS = 16384
ROW_DIM = 64
NUM_KEEP = max(1, int(NUM_ROWS * (1.0 - DROP_RATE)))

NUM_CORES = 2
NUM_SUBCORES = 16
NUM_WORKERS = NUM_CORES * NUM_SUBCORES
ROWS_PER_WORKER = ROW_DIM // NUM_WORKERS

LANES = 16
SUBLANES = 8
TILE_MINOR = 128
IN_TILES = NUM_ROWS // TILE_MINOR
OUT_TILES = -(-NUM_KEEP // TILE_MINOR)
KEEP_PAD = OUT_TILES * TILE_MINOR
NUM_STEPS = KEEP_PAD // LANES


@functools.lru_cache(maxsize=1)
def _gather_indices() -> np.ndarray:
    with jax.ensure_compile_time_eval():
        perm = jax.random.permutation(jax.random.key(42), NUM_ROWS)
    idx = np.asarray(perm)[:NUM_KEEP].astype(np.int32)
    return np.concatenate([idx, np.full(KEEP_PAD - NUM_KEEP, idx[-1], np.int32)])


@functools.lru_cache(maxsize=1)
def _build_gather():
    mesh = plsc.VectorSubcoreMesh(core_axis_name="c", subcore_axis_name="s")

    @functools.partial(
        pl.kernel,
        out_type=jax.ShapeDtypeStruct(
            (ROW_DIM // SUBLANES, OUT_TILES, SUBLANES, TILE_MINOR), jnp.float32
        ),
        mesh=mesh,
        compiler_params=pltpu.CompilerParams(
            use_tc_tiling_on_sc=False, needs_layout_passes=False
        ),
        scratch_types=[
            pltpu.VMEM((KEEP_PAD,), jnp.int32),
            pltpu.VMEM((ROWS_PER_WORKER, IN_TILES, TILE_MINOR), jnp.float32),
            pltpu.VMEM((ROWS_PER_WORKER, OUT_TILES, TILE_MINOR), jnp.float32),
            pltpu.SemaphoreType.DMA,
        ],
    )
    def _gather_cols(z_hbm, idx_hbm, y_hbm, idx_v, tbl_v, res_v, sem):
        wid = lax.axis_index("s") * NUM_CORES + lax.axis_index("c")
        copies = [pltpu.async_copy(idx_hbm, idx_v, sem)]
        for r in range(ROWS_PER_WORKER):
            row = wid * ROWS_PER_WORKER + r
            copies.append(
                pltpu.async_copy(
                    z_hbm.at[row // SUBLANES, :, row % SUBLANES, :],
                    tbl_v.at[r],
                    sem,
                )
            )
        for cp in copies:
            cp.wait()

        def step(p):
            cols = idx_v[pl.ds(p * LANES, LANES)]
            b = cols >> 7
            c = cols & (TILE_MINOR - 1)
            q = p // SUBLANES
            off = (p % SUBLANES) * LANES
            for r in range(ROWS_PER_WORKER):
                rows = jnp.full((LANES,), r, jnp.int32)
                res_v[r, q, pl.ds(off, LANES)] = plsc.load_gather(
                    tbl_v, [rows, b, c]
                )

        plsc.parallel_loop(0, NUM_STEPS, 1, unroll=16, carry=None)(step)

        for r in range(ROWS_PER_WORKER):
            row = wid * ROWS_PER_WORKER + r
            pltpu.sync_copy(
                res_v.at[r],
                y_hbm.at[row // SUBLANES, :, row % SUBLANES, :],
            )

    return _gather_cols


def kernel(instances):
    z = instances.T.reshape(
        ROW_DIM // SUBLANES, SUBLANES, IN_TILES, TILE_MINOR
    ).transpose(0, 2, 1, 3)
    y = _build_gather()(z, jnp.asarray(_gather_indices()))
    out_t = y.transpose(0, 2, 1, 3).reshape(ROW_DIM, KEEP_PAD)
    return out_t[:, :NUM_KEEP].T


# speedup vs baseline: 1.0651x; 1.0351x over previous
import functools

import jax
import jax.numpy as jnp
import numpy as np
from jax import lax
from jax.experimental import pallas as pl
from jax.experimental.pallas import tpu as pltpu
from jax.experimental.pallas import tpu_sc as plsc

DROP_RATE = 0.15
NUM_ROWS = 16384
ROW_DIM = 64
NUM_KEEP = max(1, int(NUM_ROWS * (1.0 - DROP_RATE)))

NUM_CORES = 2
NUM_SUBCORES = 16
NUM_WORKERS = NUM_CORES * NUM_SUBCORES
ROWS_PER_WORKER = ROW_DIM // NUM_WORKERS

LANES = 16
SUBLANES = 8
TILE_MINOR = 128
IN_TILES = NUM_ROWS // TILE_MINOR
OUT_TILES = -(-NUM_KEEP // TILE_MINOR)
KEEP_PAD = OUT_TILES * TILE_MINOR
NUM_BLOCKS = KEEP_PAD // (2 * LANES)
PHASE1_BLOCKS = 216
PHASE1_TILES = PHASE1_BLOCKS * 2 * LANES // TILE_MINOR


@functools.lru_cache(maxsize=1)
def _gather_indices() -> np.ndarray:
    with jax.ensure_compile_time_eval():
        perm = jax.random.permutation(jax.random.key(42), NUM_ROWS)
    idx = np.asarray(perm)[:NUM_KEEP].astype(np.int32)
    idx = np.concatenate([idx, np.full(KEEP_PAD - NUM_KEEP, idx[-1], np.int32)])
    return (
        idx.reshape(NUM_BLOCKS, 2, LANES).transpose(0, 2, 1).reshape(-1)
        .astype(np.int16)
    )


@functools.lru_cache(maxsize=1)
def _build_gather():
    mesh = plsc.VectorSubcoreMesh(core_axis_name="c", subcore_axis_name="s")

    @functools.partial(
        pl.kernel,
        out_type=jax.ShapeDtypeStruct(
            (ROW_DIM // SUBLANES, OUT_TILES, SUBLANES, TILE_MINOR), jnp.float32
        ),
        mesh=mesh,
        compiler_params=pltpu.CompilerParams(
            use_tc_tiling_on_sc=False, needs_layout_passes=False
        ),
        scratch_types=[
            pltpu.VMEM((KEEP_PAD,), jnp.int16),
            pltpu.VMEM((IN_TILES, ROWS_PER_WORKER, TILE_MINOR), jnp.float32),
            pltpu.VMEM((ROWS_PER_WORKER, OUT_TILES, TILE_MINOR), jnp.float32),
            pltpu.SemaphoreType.DMA,
            pltpu.SemaphoreType.DMA,
        ],
    )
    def _gather_cols(z_hbm, idx_hbm, y_hbm, idx_v, tbl_v, res_v, lsem, ssem):
        wid = lax.axis_index("s") * NUM_CORES + lax.axis_index("c")
        row0 = wid * ROWS_PER_WORKER
        a = row0 // SUBLANES
        rr = row0 % SUBLANES
        cp_idx = pltpu.async_copy(idx_hbm, idx_v, lsem)
        cp_tbl = pltpu.async_copy(
            z_hbm.at[a, :, pl.ds(rr, ROWS_PER_WORKER), :], tbl_v, lsem
        )
        cp_idx.wait()
        cp_tbl.wait()

        def block(k):
            v16 = idx_v[pl.ds(k * 2 * LANES, 2 * LANES)]
            halves = plsc.unpack(v16, format=plsc.PackFormat.INTERLEAVED)
            for half, cols in enumerate(halves):
                p = 2 * k + half
                q = p // SUBLANES
                off = (p % SUBLANES) * LANES
                b = cols >> 7
                c = cols & (TILE_MINOR - 1)
                for r in range(ROWS_PER_WORKER):
                    rows = jnp.full((LANES,), r, jnp.int32)
                    res_v[r, q, pl.ds(off, LANES)] = plsc.load_gather(
                        tbl_v, [b, rows, c]
                    )

        plsc.parallel_loop(0, PHASE1_BLOCKS, 1, unroll=8, carry=None)(block)
        stores = [
            pltpu.async_copy(
                res_v.at[r, pl.ds(0, PHASE1_TILES)],
                y_hbm.at[a, pl.ds(0, PHASE1_TILES), rr + r, :],
                ssem,
            )
            for r in range(ROWS_PER_WORKER)
        ]
        plsc.parallel_loop(PHASE1_BLOCKS, NUM_BLOCKS, 1, unroll=8, carry=None)(block)
        for r in range(ROWS_PER_WORKER):
            stores.append(
                pltpu.async_copy(
                    res_v.at[r, pl.ds(PHASE1_TILES, OUT_TILES - PHASE1_TILES)],
                    y_hbm.at[a, pl.ds(PHASE1_TILES, OUT_TILES - PHASE1_TILES), rr + r, :],
                    ssem,
                )
            )
        for s in stores:
            s.wait()

    return _gather_cols


def kernel(instances):
    z = instances.T.reshape(
        ROW_DIM // SUBLANES, SUBLANES, IN_TILES, TILE_MINOR
    ).transpose(0, 2, 1, 3)
    y = _build_gather()(z, jnp.asarray(_gather_indices()))
    out_t = y.transpose(0, 2, 1, 3).reshape(ROW_DIM, KEEP_PAD)
    return out_t[:, :NUM_KEEP].T
